# Initial kernel scaffold; baseline (speedup 1.0000x reference)
import jax
import jax.numpy as jnp
from jax.experimental import pallas as pl


def kernel(v, e, variables, edge_index, W1e, b1e, W2e, b2e, g_e, be_e, W1n, b1n, W2n, b2n, g_n, be_n):
    raise NotImplementedError("write your pallas kernel here")


# speedup vs baseline: 3.2401x; 3.2401x over previous
import functools
import math

import jax
import jax.numpy as jnp
from jax import lax
from jax.experimental import pallas as pl
from jax.experimental.pallas import tpu as pltpu
from jax.experimental.pallas import tpu_sc as plsc

_NC, _NS = 2, 16
_CH = 80
_CW = 16


def _blk(n, prefs):
    for b in prefs:
        if n % b == 0:
            return b
    return n


def _swish(x):
    return x * jax.nn.sigmoid(x)


def _ln(x, g, b, eps=1e-5):
    m = jnp.mean(x, axis=-1, keepdims=True)
    d = x - m
    var = jnp.mean(d * d, axis=-1, keepdims=True)
    return d * lax.rsqrt(var + eps) * g + b


def _node_proj(v, wa, wb):
    N, F = v.shape
    H = wa.shape[1]
    BN = _blk(N, [1000, 400, 200, 80, 40, 16, 8])
    out = jax.ShapeDtypeStruct((N, H), jnp.float32)

    def body(v_ref, wa_ref, wb_ref, a_ref, b_ref):
        vb = v_ref[...]
        a_ref[...] = jnp.dot(vb, wa_ref[...], preferred_element_type=jnp.float32)
        b_ref[...] = jnp.dot(vb, wb_ref[...], preferred_element_type=jnp.float32)

    return pl.pallas_call(
        body,
        grid=(N // BN,),
        in_specs=[
            pl.BlockSpec((BN, F), lambda i: (i, 0)),
            pl.BlockSpec((F, H), lambda i: (0, 0)),
            pl.BlockSpec((F, H), lambda i: (0, 0)),
        ],
        out_specs=[
            pl.BlockSpec((BN, H), lambda i: (i, 0)),
            pl.BlockSpec((BN, H), lambda i: (i, 0)),
        ],
        out_shape=[out, out],
    )(v, wa, wb)


def _edge_mlp(G, e, we, b1, w2, b2, gam, bet):
    E, H = G.shape
    BE = _blk(E, [2000, 1000, 400, 160, 80, 40, 16, 8])

    def body(g_ref, e_ref, we_ref, b1_ref, w2_ref, b2_ref, gm_ref, bt_ref, o_ref):
        eb = e_ref[...]
        x = g_ref[...] + b1_ref[...]
        x = x + eb[:, 0:1] * we_ref[0:1, :] + eb[:, 1:2] * we_ref[1:2, :]
        x = _swish(x)
        y = _swish(jnp.dot(x, w2_ref[...], preferred_element_type=jnp.float32)
                   + b2_ref[...])
        o_ref[...] = _ln(y, gm_ref[...], bt_ref[...])

    return pl.pallas_call(
        body,
        grid=(E // BE,),
        in_specs=[
            pl.BlockSpec((BE, H), lambda i: (i, 0)),
            pl.BlockSpec((BE, 2), lambda i: (i, 0)),
            pl.BlockSpec((2, H), lambda i: (0, 0)),
            pl.BlockSpec((1, H), lambda i: (0, 0)),
            pl.BlockSpec((H, H), lambda i: (0, 0)),
            pl.BlockSpec((1, H), lambda i: (0, 0)),
            pl.BlockSpec((1, H), lambda i: (0, 0)),
            pl.BlockSpec((1, H), lambda i: (0, 0)),
        ],
        out_specs=pl.BlockSpec((BE, H), lambda i: (i, 0)),
        out_shape=jax.ShapeDtypeStruct((E, H), jnp.float32),
    )(G, e, we, b1.reshape(1, H), w2, b2.reshape(1, H),
      gam.reshape(1, H), bet.reshape(1, H))


def _node_mlp(ssum_p, cnt_p, v, variables, w1a, w1v, w1var, b1, w2, b2, gam, bet):
    N, F = v.shape
    H = w2.shape[0]
    NV = variables.shape[1]
    BN = _blk(N, [1000, 400, 200, 80, 40, 16, 8])

    def body(s_ref, c_ref, v_ref, var_ref, w1a_ref, w1v_ref, w1var_ref,
             b1_ref, w2_ref, b2_ref, gm_ref, bt_ref, o_ref):
        s = s_ref[0] + s_ref[1]
        cnt = c_ref[0, :, 0:1] + c_ref[1, :, 0:1]
        aggr = s / jnp.maximum(cnt, 1.0)
        x = jnp.dot(aggr, w1a_ref[...], preferred_element_type=jnp.float32)
        x = x + jnp.dot(v_ref[...], w1v_ref[...], preferred_element_type=jnp.float32)
        varb = var_ref[...]
        for k in range(NV):
            x = x + varb[:, k:k + 1] * w1var_ref[k:k + 1, :]
        x = _swish(x + b1_ref[...])
        y = _swish(jnp.dot(x, w2_ref[...], preferred_element_type=jnp.float32)
                   + b2_ref[...])
        o_ref[...] = _ln(y, gm_ref[...], bt_ref[...])

    return pl.pallas_call(
        body,
        grid=(N // BN,),
        in_specs=[
            pl.BlockSpec((_NC, BN, H), lambda i: (0, i, 0)),
            pl.BlockSpec((_NC, BN, H), lambda i: (0, i, 0)),
            pl.BlockSpec((BN, F), lambda i: (i, 0)),
            pl.BlockSpec((BN, NV), lambda i: (i, 0)),
            pl.BlockSpec((H, H), lambda i: (0, 0)),
            pl.BlockSpec((F, H), lambda i: (0, 0)),
            pl.BlockSpec((NV, H), lambda i: (0, 0)),
            pl.BlockSpec((1, H), lambda i: (0, 0)),
            pl.BlockSpec((H, H), lambda i: (0, 0)),
            pl.BlockSpec((1, H), lambda i: (0, 0)),
            pl.BlockSpec((1, H), lambda i: (0, 0)),
            pl.BlockSpec((1, H), lambda i: (0, 0)),
        ],
        out_specs=pl.BlockSpec((BN, H), lambda i: (i, 0)),
        out_shape=jax.ShapeDtypeStruct((N, H), jnp.float32),
    )(ssum_p, cnt_p, v, variables, w1a, w1v, w1var, b1.reshape(1, H), w2,
      b2.reshape(1, H), gam.reshape(1, H), bet.reshape(1, H))


def _edge_gather(A, B, row, col):
    N, H = A.shape
    E = row.shape[0]
    NW = _NC * _NS
    assert E % (NW * _CH) == 0
    EPW = E // NW
    NCH = EPW // _CH
    mesh = plsc.VectorSubcoreMesh(core_axis_name="c", subcore_axis_name="s",
                                  num_cores=_NC, num_subcores=_NS)

    @functools.partial(
        pl.kernel,
        out_type=jax.ShapeDtypeStruct((E, H), jnp.float32),
        mesh=mesh,
        scratch_types=[
            pltpu.VMEM((_CH,), jnp.int32),
            pltpu.VMEM((_CH,), jnp.int32),
            pltpu.VMEM((_CH, H), jnp.float32),
            pltpu.VMEM((_CH, H), jnp.float32),
            pltpu.SemaphoreType.DMA,
            pltpu.SemaphoreType.DMA,
        ],
    )
    def gather_k(a_hbm, b_hbm, row_hbm, col_hbm, g_hbm,
                 idx1, idx2, buf1, buf2, sem1, sem2):
        wid = lax.axis_index("s") * _NC + lax.axis_index("c")
        base = wid * EPW

        def chunk(c, carry):
            off = base + c * _CH
            pltpu.sync_copy(row_hbm.at[pl.ds(off, _CH)], idx1)
            pltpu.sync_copy(col_hbm.at[pl.ds(off, _CH)], idx2)
            cp1 = pltpu.async_copy(a_hbm.at[idx1], buf1, sem1)
            cp2 = pltpu.async_copy(b_hbm.at[idx2], buf2, sem2)
            cp1.wait()
            cp2.wait()

            def addrow(i, carry2):
                for j in range(H // 16):
                    sl = pl.ds(j * 16, 16)
                    buf1[i, sl] = buf1[i, sl] + buf2[i, sl]
                return carry2

            lax.fori_loop(0, _CH, addrow, 0)
            pltpu.sync_copy(buf1, g_hbm.at[pl.ds(off, _CH)])
            return carry

        lax.fori_loop(0, NCH, chunk, 0)

    return gather_k(A, B, row, col)


def _scatter_mean_parts(e_out, col, N):
    E, H = e_out.shape
    NW = _NC * _NS
    assert E % (NW * _CH) == 0 and N % _CH == 0
    EPW = E // NW
    NCH = EPW // _CH
    NRC = N // _CH
    KMAX = -(-NRC // _NS)
    mesh = plsc.VectorSubcoreMesh(core_axis_name="c", subcore_axis_name="s",
                                  num_cores=_NC, num_subcores=_NS)

    @functools.partial(
        pl.kernel,
        out_type=(jax.ShapeDtypeStruct((_NC, N, H), jnp.float32),
                  jax.ShapeDtypeStruct((_NC, N, H), jnp.float32)),
        mesh=mesh,
        scratch_types=[
            pltpu.VMEM((_CH,), jnp.int32),
            pltpu.VMEM((_CH, H), jnp.float32),
            pltpu.VMEM_SHARED((N, H), jnp.float32),
        ],
    )
    def scatter_k(eout_hbm, col_hbm, ssum_hbm, cnt_hbm, idx, ebuf, acc_sh):
        cid = lax.axis_index("c")
        sid = lax.axis_index("s")
        wid = sid * _NC + cid

        def fill(val):
            def body(i, carry):
                for j in range(H // 16):
                    ebuf[i, pl.ds(j * 16, 16)] = jnp.full((16,), val, jnp.float32)
                return carry
            lax.fori_loop(0, _CH, body, 0)

        def zero_acc():
            def zchunk(k, carry):
                c = sid + k * _NS

                @pl.when(c < NRC)
                def _():
                    pltpu.sync_copy(ebuf, acc_sh.at[pl.ds(c * _CH, _CH)])
                return carry
            lax.fori_loop(0, KMAX, zchunk, 0)

        def copy_out(dst):
            def ochunk(k, carry):
                c = sid + k * _NS

                @pl.when(c < NRC)
                def _():
                    pltpu.sync_copy(acc_sh.at[pl.ds(c * _CH, _CH)],
                                    dst.at[cid, pl.ds(c * _CH, _CH)])
                return carry
            lax.fori_loop(0, KMAX, ochunk, 0)

        fill(0.0)
        zero_acc()
        plsc.subcore_barrier()

        def chunkA(c, carry):
            off = wid * EPW + c * _CH
            pltpu.sync_copy(col_hbm.at[pl.ds(off, _CH)], idx)
            pltpu.sync_copy(eout_hbm.at[pl.ds(off, _CH)], ebuf)
            pltpu.sync_copy(ebuf, acc_sh.at[idx], add=True)
            return carry

        lax.fori_loop(0, NCH, chunkA, 0)
        plsc.subcore_barrier()
        copy_out(ssum_hbm)
        plsc.subcore_barrier()

        fill(0.0)
        zero_acc()
        plsc.subcore_barrier()
        fill(1.0)

        def chunkB(c, carry):
            off = wid * EPW + c * _CH
            pltpu.sync_copy(col_hbm.at[pl.ds(off, _CH)], idx)
            pltpu.sync_copy(ebuf, acc_sh.at[idx], add=True)
            return carry

        lax.fori_loop(0, NCH, chunkB, 0)
        plsc.subcore_barrier()
        copy_out(cnt_hbm)

    return scatter_k(e_out, col)


def kernel(v, e, variables, edge_index, W1e, b1e, W2e, b2e, g_e, be_e,
           W1n, b1n, W2n, b2n, g_n, be_n):
    N, F = v.shape
    E_ = e.shape[0]
    H = W2e.shape[0]
    row = edge_index[0]
    col = edge_index[1]
    wa = W1e[2:2 + F]
    wb = W1e[2 + F:2 + 2 * F]
    we = W1e[:2]
    w1a = W1n[:H]
    w1v = W1n[H:H + F]
    w1var = W1n[H + F:]

    A, B = _node_proj(v, wa, wb)
    G = _edge_gather(A, B, row, col)
    e_out = _edge_mlp(G, e, we, b1e, W2e, b2e, g_e, be_e)
    ssum_p, cnt_p = _scatter_mean_parts(e_out, col, N)
    v_out = _node_mlp(ssum_p, cnt_p, v, variables, w1a, w1v, w1var,
                      b1n, W2n, b2n, g_n, be_n)
    return (v_out, e_out)


# speedup vs baseline: 5.4557x; 1.6838x over previous
import functools
import math

import jax
import jax.numpy as jnp
from jax import lax
from jax.experimental import pallas as pl
from jax.experimental.pallas import tpu as pltpu
from jax.experimental.pallas import tpu_sc as plsc

_NC, _NS = 2, 16
_CH = 80
_CW = 16


def _blk(n, prefs):
    for b in prefs:
        if n % b == 0:
            return b
    return n


def _swish(x):
    return x * jax.nn.sigmoid(x)


def _ln(x, g, b, eps=1e-5):
    m = jnp.mean(x, axis=-1, keepdims=True)
    d = x - m
    var = jnp.mean(d * d, axis=-1, keepdims=True)
    return d * lax.rsqrt(var + eps) * g + b


def _node_proj(v, wa, wb):
    N, F = v.shape
    H = wa.shape[1]
    BN = _blk(N, [1000, 400, 200, 80, 40, 16, 8])
    out = jax.ShapeDtypeStruct((N, H), jnp.float32)

    def body(v_ref, wa_ref, wb_ref, a_ref, b_ref):
        vb = v_ref[...]
        a_ref[...] = jnp.dot(vb, wa_ref[...], preferred_element_type=jnp.float32)
        b_ref[...] = jnp.dot(vb, wb_ref[...], preferred_element_type=jnp.float32)

    return pl.pallas_call(
        body,
        grid=(N // BN,),
        in_specs=[
            pl.BlockSpec((BN, F), lambda i: (i, 0)),
            pl.BlockSpec((F, H), lambda i: (0, 0)),
            pl.BlockSpec((F, H), lambda i: (0, 0)),
        ],
        out_specs=[
            pl.BlockSpec((BN, H), lambda i: (i, 0)),
            pl.BlockSpec((BN, H), lambda i: (i, 0)),
        ],
        out_shape=[out, out],
    )(v, wa, wb)


def _edge_mlp(G, e, we, b1, w2, b2, gam, bet):
    E, H = G.shape
    BE = _blk(E, [2000, 1000, 400, 160, 80, 40, 16, 8])

    def body(g_ref, e_ref, we_ref, b1_ref, w2_ref, b2_ref, gm_ref, bt_ref, o_ref):
        eb = e_ref[...]
        x = g_ref[...] + b1_ref[...]
        x = x + eb[:, 0:1] * we_ref[0:1, :] + eb[:, 1:2] * we_ref[1:2, :]
        x = _swish(x)
        y = _swish(jnp.dot(x, w2_ref[...], preferred_element_type=jnp.float32)
                   + b2_ref[...])
        o_ref[...] = _ln(y, gm_ref[...], bt_ref[...])

    return pl.pallas_call(
        body,
        grid=(E // BE,),
        in_specs=[
            pl.BlockSpec((BE, H), lambda i: (i, 0)),
            pl.BlockSpec((BE, 2), lambda i: (i, 0)),
            pl.BlockSpec((2, H), lambda i: (0, 0)),
            pl.BlockSpec((1, H), lambda i: (0, 0)),
            pl.BlockSpec((H, H), lambda i: (0, 0)),
            pl.BlockSpec((1, H), lambda i: (0, 0)),
            pl.BlockSpec((1, H), lambda i: (0, 0)),
            pl.BlockSpec((1, H), lambda i: (0, 0)),
        ],
        out_specs=pl.BlockSpec((BE, H), lambda i: (i, 0)),
        out_shape=jax.ShapeDtypeStruct((E, H), jnp.float32),
    )(G, e, we, b1.reshape(1, H), w2, b2.reshape(1, H),
      gam.reshape(1, H), bet.reshape(1, H))


def _node_mlp(ssum_p, cnt_p, v, variables, w1a, w1v, w1var, b1, w2, b2, gam, bet):
    N, F = v.shape
    H = w2.shape[0]
    NV = variables.shape[1]
    BN = _blk(N, [1000, 400, 200, 80, 40, 16, 8])

    def body(s_ref, c_ref, v_ref, var_ref, w1a_ref, w1v_ref, w1var_ref,
             b1_ref, w2_ref, b2_ref, gm_ref, bt_ref, o_ref):
        s = s_ref[0] + s_ref[1]
        cnt = c_ref[0, :, 0:1] + c_ref[1, :, 0:1]
        aggr = s / jnp.maximum(cnt, 1.0)
        x = jnp.dot(aggr, w1a_ref[...], preferred_element_type=jnp.float32)
        x = x + jnp.dot(v_ref[...], w1v_ref[...], preferred_element_type=jnp.float32)
        varb = var_ref[...]
        for k in range(NV):
            x = x + varb[:, k:k + 1] * w1var_ref[k:k + 1, :]
        x = _swish(x + b1_ref[...])
        y = _swish(jnp.dot(x, w2_ref[...], preferred_element_type=jnp.float32)
                   + b2_ref[...])
        o_ref[...] = _ln(y, gm_ref[...], bt_ref[...])

    return pl.pallas_call(
        body,
        grid=(N // BN,),
        in_specs=[
            pl.BlockSpec((_NC, BN, H), lambda i: (0, i, 0)),
            pl.BlockSpec((_NC, BN, H), lambda i: (0, i, 0)),
            pl.BlockSpec((BN, F), lambda i: (i, 0)),
            pl.BlockSpec((BN, NV), lambda i: (i, 0)),
            pl.BlockSpec((H, H), lambda i: (0, 0)),
            pl.BlockSpec((F, H), lambda i: (0, 0)),
            pl.BlockSpec((NV, H), lambda i: (0, 0)),
            pl.BlockSpec((1, H), lambda i: (0, 0)),
            pl.BlockSpec((H, H), lambda i: (0, 0)),
            pl.BlockSpec((1, H), lambda i: (0, 0)),
            pl.BlockSpec((1, H), lambda i: (0, 0)),
            pl.BlockSpec((1, H), lambda i: (0, 0)),
        ],
        out_specs=pl.BlockSpec((BN, H), lambda i: (i, 0)),
        out_shape=jax.ShapeDtypeStruct((N, H), jnp.float32),
    )(ssum_p, cnt_p, v, variables, w1a, w1v, w1var, b1.reshape(1, H), w2,
      b2.reshape(1, H), gam.reshape(1, H), bet.reshape(1, H))


def _edge_gather(A, B, row3d, col3d):
    N, H = A.shape
    NW, NCH, CH = row3d.shape
    assert NW == _NC * _NS and CH == _CH
    E = NW * NCH * CH
    EPW = NCH * CH
    mesh = plsc.VectorSubcoreMesh(core_axis_name="c", subcore_axis_name="s",
                                  num_cores=_NC, num_subcores=_NS)

    @functools.partial(
        pl.kernel,
        out_type=jax.ShapeDtypeStruct((E, H), jnp.float32),
        mesh=mesh,
        scratch_types=[
            pltpu.VMEM((NCH, CH), jnp.int32),
            pltpu.VMEM((NCH, CH), jnp.int32),
            pltpu.VMEM((2, CH, H), jnp.float32),
            pltpu.VMEM((2, CH, H), jnp.float32),
            pltpu.SemaphoreType.DMA,
            pltpu.SemaphoreType.DMA,
        ],
    )
    def gather_k(a_hbm, b_hbm, row_hbm, col_hbm, g_hbm,
                 idxr, idxc, buf1, buf2, semA, semB):
        wid = lax.axis_index("s") * _NC + lax.axis_index("c")
        base = wid * EPW
        pltpu.sync_copy(row_hbm.at[wid], idxr)
        pltpu.sync_copy(col_hbm.at[wid], idxc)
        sems = [semA, semB]

        def start(c, b):
            pltpu.async_copy(a_hbm.at[idxr.at[c]], buf1.at[b], sems[b])
            pltpu.async_copy(b_hbm.at[idxc.at[c]], buf2.at[b], sems[b])

        def drain(c, b):
            pltpu.make_async_copy(a_hbm.at[idxr.at[c]], buf1.at[b],
                                  sems[b]).wait()
            pltpu.make_async_copy(b_hbm.at[idxc.at[c]], buf2.at[b],
                                  sems[b]).wait()

        def process(c, b):
            def addrow(i, carry2):
                for j in range(H // 16):
                    sl = pl.ds(j * 16, 16)
                    buf1[b, i, sl] = buf1[b, i, sl] + buf2[b, i, sl]
                return carry2

            lax.fori_loop(0, CH, addrow, 0)
            pltpu.sync_copy(buf1.at[b], g_hbm.at[pl.ds(base + c * CH, CH)])

        start(0, 0)
        if NCH > 1:
            start(1, 1)

        def body(g, carry):
            c0 = 2 * g
            drain(c0, 0)
            process(c0, 0)

            @pl.when(c0 + 2 < NCH)
            def _():
                start(c0 + 2, 0)

            @pl.when(c0 + 1 < NCH)
            def _():
                drain(c0 + 1, 1)
                process(c0 + 1, 1)

            @pl.when(c0 + 3 < NCH)
            def _():
                start(c0 + 3, 1)

            return carry

        lax.fori_loop(0, -(-NCH // 2), body, 0)

    return gather_k(A, B, row3d, col3d)


def _scatter_sum(e_out, col3d, N):
    E, H = e_out.shape
    NW, NCH, CH = col3d.shape
    assert NW == _NC * _NS and CH == _CH and N % _CH == 0
    EPW = NCH * CH
    NRC = N // _CH
    KMAX = -(-NRC // _NS)
    mesh = plsc.VectorSubcoreMesh(core_axis_name="c", subcore_axis_name="s",
                                  num_cores=_NC, num_subcores=_NS)

    @functools.partial(
        pl.kernel,
        out_type=jax.ShapeDtypeStruct((_NC, N, H), jnp.float32),
        mesh=mesh,
        scratch_types=[
            pltpu.VMEM((NCH, _CH), jnp.int32),
            pltpu.VMEM((2, _CH, H), jnp.float32),
            pltpu.VMEM_SHARED((N, H), jnp.float32),
            pltpu.SemaphoreType.DMA,
            pltpu.SemaphoreType.DMA,
        ],
    )
    def scatter_k(eout_hbm, col_hbm, ssum_hbm, idx2d, ebuf, acc_sh,
                  semA, semB):
        cid = lax.axis_index("c")
        sid = lax.axis_index("s")
        wid = sid * _NC + cid
        base = wid * EPW
        sems = [semA, semB]

        def fill(val):
            def body(i, carry):
                for j in range(H // 16):
                    ebuf[0, i, pl.ds(j * 16, 16)] = jnp.full((16,), val,
                                                             jnp.float32)
                return carry
            lax.fori_loop(0, _CH, body, 0)

        def zero_acc():
            def zchunk(k, carry):
                c = sid + k * _NS

                @pl.when(c < NRC)
                def _():
                    pltpu.sync_copy(ebuf.at[0], acc_sh.at[pl.ds(c * _CH, _CH)])
                return carry
            lax.fori_loop(0, KMAX, zchunk, 0)

        def copy_out(dst):
            def ochunk(k, carry):
                c = sid + k * _NS

                @pl.when(c < NRC)
                def _():
                    pltpu.sync_copy(acc_sh.at[pl.ds(c * _CH, _CH)],
                                    dst.at[cid, pl.ds(c * _CH, _CH)])
                return carry
            lax.fori_loop(0, KMAX, ochunk, 0)

        fill(0.0)
        zero_acc()
        pltpu.sync_copy(col_hbm.at[wid], idx2d)
        plsc.subcore_barrier()

        def start(c, b):
            pltpu.async_copy(eout_hbm.at[pl.ds(base + c * _CH, _CH)],
                             ebuf.at[b], sems[b])

        def drain(c, b):
            pltpu.make_async_copy(eout_hbm.at[pl.ds(base + c * _CH, _CH)],
                                  ebuf.at[b], sems[b]).wait()

        def scat(c, b):
            pltpu.sync_copy(ebuf.at[b], acc_sh.at[idx2d.at[c]], add=True)

        start(0, 0)
        if NCH > 1:
            start(1, 1)

        def body(g, carry):
            c0 = 2 * g
            drain(c0, 0)
            scat(c0, 0)

            @pl.when(c0 + 2 < NCH)
            def _():
                start(c0 + 2, 0)

            @pl.when(c0 + 1 < NCH)
            def _():
                drain(c0 + 1, 1)
                scat(c0 + 1, 1)

            @pl.when(c0 + 3 < NCH)
            def _():
                start(c0 + 3, 1)

            return carry

        lax.fori_loop(0, -(-NCH // 2), body, 0)
        plsc.subcore_barrier()
        copy_out(ssum_hbm)

    return scatter_k(e_out, col3d)


def _count_scatter(col3d, N, H):
    NW, NCH, CH = col3d.shape
    assert NW == _NC * _NS and CH == _CH and N % _CH == 0
    NRC = N // _CH
    KMAX = -(-NRC // _NS)
    mesh = plsc.VectorSubcoreMesh(core_axis_name="c", subcore_axis_name="s",
                                  num_cores=_NC, num_subcores=_NS)

    @functools.partial(
        pl.kernel,
        out_type=jax.ShapeDtypeStruct((_NC, N, H), jnp.float32),
        mesh=mesh,
        scratch_types=[
            pltpu.VMEM((NCH, _CH), jnp.int32),
            pltpu.VMEM((_CH, H), jnp.float32),
            pltpu.VMEM_SHARED((N, H), jnp.float32),
        ],
    )
    def count_k(col_hbm, cnt_hbm, idx2d, onesbuf, acc_sh):
        cid = lax.axis_index("c")
        sid = lax.axis_index("s")
        wid = sid * _NC + cid

        def fill(val):
            def body(i, carry):
                for j in range(H // 16):
                    onesbuf[i, pl.ds(j * 16, 16)] = jnp.full((16,), val,
                                                             jnp.float32)
                return carry
            lax.fori_loop(0, _CH, body, 0)

        fill(0.0)

        def zchunk(k, carry):
            c = sid + k * _NS

            @pl.when(c < NRC)
            def _():
                pltpu.sync_copy(onesbuf, acc_sh.at[pl.ds(c * _CH, _CH)])
            return carry
        lax.fori_loop(0, KMAX, zchunk, 0)
        pltpu.sync_copy(col_hbm.at[wid], idx2d)
        fill(1.0)
        plsc.subcore_barrier()

        def chunk(c, carry):
            pltpu.sync_copy(onesbuf, acc_sh.at[idx2d.at[c]], add=True)
            return carry

        lax.fori_loop(0, NCH, chunk, 0)
        plsc.subcore_barrier()

        def ochunk(k, carry):
            c = sid + k * _NS

            @pl.when(c < NRC)
            def _():
                pltpu.sync_copy(acc_sh.at[pl.ds(c * _CH, _CH)],
                                cnt_hbm.at[cid, pl.ds(c * _CH, _CH)])
            return carry
        lax.fori_loop(0, KMAX, ochunk, 0)

    return count_k(col3d)


def kernel(v, e, variables, edge_index, W1e, b1e, W2e, b2e, g_e, be_e,
           W1n, b1n, W2n, b2n, g_n, be_n):
    N, F = v.shape
    E_ = e.shape[0]
    H = W2e.shape[0]
    NW = _NC * _NS
    NCH = E_ // (NW * _CH)
    row3d = edge_index[0].reshape(NW, NCH, _CH)
    col3d = edge_index[1].reshape(NW, NCH, _CH)
    wa = W1e[2:2 + F]
    wb = W1e[2 + F:2 + 2 * F]
    we = W1e[:2]
    w1a = W1n[:H]
    w1v = W1n[H:H + F]
    w1var = W1n[H + F:]

    A, B = _node_proj(v, wa, wb)
    G = _edge_gather(A, B, row3d, col3d)
    e_out = _edge_mlp(G, e, we, b1e, W2e, b2e, g_e, be_e)
    cnt_p = _count_scatter(col3d, N, H)
    ssum_p = _scatter_sum(e_out, col3d, N)
    v_out = _node_mlp(ssum_p, cnt_p, v, variables, w1a, w1v, w1var,
                      b1n, W2n, b2n, g_n, be_n)
    return (v_out, e_out)
